# Initial kernel scaffold; baseline (speedup 1.0000x reference)
import jax
import jax.numpy as jnp
from jax.experimental import pallas as pl


def kernel(node_feat, coord, edge_index, We1, be1, We2, be2, Wn1, bn1, Wn2, bn2, Wc1, bc1, Wc2):
    raise NotImplementedError("write your pallas kernel here")


# speedup vs baseline: 5.4688x; 5.4688x over previous
import jax
import jax.numpy as jnp
from jax import lax
from jax.experimental import pallas as pl
from jax.experimental.pallas import tpu as pltpu
from jax.experimental.pallas import tpu_sc as plsc

N = 10000
E = 320000
D = 128
CD = 3
NC = 2
NS = 16
NW = NC * NS
L = 16
CHUNK = 128
NCHUNK = E // CHUNK
CPW = (NCHUNK + NW - 1) // NW
BR = 80
NB = N // BR
BPS = (NB + NS - 1) // NS
NP = 10240
TPS = 4 * NP // NS

_f32 = jnp.float32
_i32 = jnp.int32


def _silu(x):
    return x * jax.nn.sigmoid(x)


def _tables_body(nf, wa, wb, t0, t1):
    x = nf[...]
    t0[...] = jnp.dot(x, wa[...], preferred_element_type=_f32)
    t1[...] = jnp.dot(x, wb[...], preferred_element_type=_f32)


def _edge_body(g0, g1, cdt, wlast, be1, we2, be2, wc1, bc1, wc2r, ef, tt):
    dT = cdt[...][0:CD, :]
    rad = lax.dot_general(dT * dT, jnp.ones((CD, 1), _f32),
                          (((0,), (0,)), ((), ())),
                          preferred_element_type=_f32)
    h = _silu(g0[...] + g1[...] + rad * wlast[...] + be1[...])
    e = _silu(jnp.dot(h, we2[...], preferred_element_type=_f32) + be2[...])
    c = _silu(jnp.dot(e, wc1[...], preferred_element_type=_f32) + bc1[...])
    sT = lax.dot_general(wc2r[...], c, (((1,), (1,)), ((), ())),
                         preferred_element_type=_f32)
    ef[...] = e
    nb = dT.shape[1]
    tt[...] = jnp.concatenate(
        [dT * sT, jnp.ones((1, nb), _f32), jnp.zeros((4, nb), _f32)], axis=0)


def _node_body(nf, cpT, acc, tx, wn1a, wn1b, bn1, wn2, bn2, nout, coutT):
    ad = acc[...]
    agg = ad[0] + ad[1]
    t = tx[...]
    t4 = t[0:4] + t[4:8]
    cnt = jnp.maximum(t4[CD:CD + 1, :], 1.0)
    caggT = t4[0:CD, :] / cnt
    coutT[...] = cpT[...] + jnp.concatenate(
        [caggT, jnp.zeros((8 - CD, t.shape[1]), _f32)], axis=0)
    x = nf[...]
    nh = _silu(jnp.dot(x, wn1a[...], preferred_element_type=_f32)
               + jnp.dot(agg, wn1b[...], preferred_element_type=_f32)
               + bn1[...])
    nout[...] = x + jnp.dot(nh, wn2[...], preferred_element_type=_f32) + bn2[...]


def _sc_gather_body(t0, t1, i0, i1, cp, g0, g1, cdt,
                    idx0_v, idx1_v, rows0_v, rows1_v, douts_v, coordv, sem):
    cid = lax.axis_index("c")
    sid = lax.axis_index("s")
    w = sid * NC + cid
    pltpu.sync_copy(cp, coordv)

    def body(i, carry):
        c = w + i * NW

        @pl.when(c < NCHUNK)
        def _():
            base = c * CHUNK
            pltpu.sync_copy(i0.at[pl.ds(base, CHUNK)], idx0_v)
            pltpu.sync_copy(i1.at[pl.ds(base, CHUNK)], idx1_v)
            cp0 = pltpu.async_copy(t0.at[idx0_v], rows0_v, sem)
            cp1 = pltpu.async_copy(t1.at[idx1_v], rows1_v, sem)
            for g in range(CHUNK // L):
                ids0 = idx0_v[pl.ds(g * L, L)]
                ids1 = idx1_v[pl.ds(g * L, L)]
                r0 = lax.shift_right_logical(ids0, 4)
                r1 = lax.shift_right_logical(ids1, 4)
                l0 = (ids0 & 15) * 8
                l1 = (ids1 & 15) * 8
                for d in range(CD):
                    c0 = plsc.load_gather(coordv, [r0, l0 + d])
                    c1 = plsc.load_gather(coordv, [r1, l1 + d])
                    douts_v[d, pl.ds(g * L, L)] = c0 - c1
            cp0.wait()
            cp1.wait()
            pltpu.sync_copy(rows0_v, g0.at[pl.ds(base, CHUNK)])
            pltpu.sync_copy(rows1_v, g1.at[pl.ds(base, CHUNK)])
            pltpu.sync_copy(douts_v, cdt.at[:, pl.ds(base, CHUNK)])

        return carry

    lax.fori_loop(0, CPW, body, 0)


def _sc_scatter_body(ef, tt, i0, z128, z1, acc_out, t_out,
                     idx_v, idx4_v, rows_v, ttv, zbuf_v, zb1_v, acc_sh, acct_sh):
    cid = lax.axis_index("c")
    sid = lax.axis_index("s")
    w = sid * NC + cid

    pltpu.sync_copy(z128, zbuf_v)
    pltpu.sync_copy(z1, zb1_v)
    for j in range(BPS):
        b = sid + j * NS

        @pl.when(b < NB)
        def _():
            pltpu.sync_copy(zbuf_v, acc_sh.at[pl.ds(b * BR, BR)])

    pltpu.sync_copy(zb1_v, acct_sh.at[pl.ds(sid * TPS, TPS)])
    plsc.subcore_barrier()

    def body(i, carry):
        c = w + i * NW

        @pl.when(c < NCHUNK)
        def _():
            base = c * CHUNK
            pltpu.sync_copy(i0.at[pl.ds(base, CHUNK)], idx_v)
            pltpu.sync_copy(ef.at[pl.ds(base, CHUNK)], rows_v)
            pltpu.sync_copy(tt.at[:, pl.ds(base, CHUNK)], ttv)
            pltpu.sync_copy(rows_v, acc_sh.at[idx_v], add=True)
            for g in range(CHUNK // L):
                ids = idx_v[pl.ds(g * L, L)]
                for d in range(4):
                    idx4_v[d, pl.ds(g * L, L)] = ids + d * NP
            for d in range(4):
                pltpu.sync_copy(ttv.at[d], acct_sh.at[idx4_v.at[d]], add=True)

        return carry

    lax.fori_loop(0, CPW, body, 0)
    plsc.subcore_barrier()

    for j in range(BPS):
        b = sid + j * NS

        @pl.when(b < NB)
        def _():
            pltpu.sync_copy(acc_sh.at[pl.ds(b * BR, BR)], zbuf_v)
            pltpu.sync_copy(zbuf_v, acc_out.at[cid, pl.ds(b * BR, BR), :])

    pltpu.sync_copy(acct_sh.at[pl.ds(sid * TPS, TPS)], zb1_v)
    pltpu.sync_copy(zb1_v, t_out.at[cid, pl.ds(sid * TPS, TPS)])


def kernel(node_feat, coord, edge_index, We1, be1, We2, be2,
           Wn1, bn1, Wn2, bn2, Wc1, bc1, Wc2):
    ei0 = edge_index[0]
    ei1 = edge_index[1]
    coord8 = jnp.pad(coord, ((0, 0), (0, 8 - CD)))
    cp625 = jnp.reshape(coord8, (N // L, 8 * L))
    cpT = jnp.pad(coord8, ((0, NP - N), (0, 0))).T
    nfp = jnp.pad(node_feat, ((0, NP - N), (0, 0)))

    rblk = 2000
    tables = pl.pallas_call(
        _tables_body,
        grid=(N // rblk,),
        in_specs=[
            pl.BlockSpec((rblk, D), lambda i: (i, 0)),
            pl.BlockSpec((D, D), lambda i: (0, 0)),
            pl.BlockSpec((D, D), lambda i: (0, 0)),
        ],
        out_specs=(pl.BlockSpec((rblk, D), lambda i: (i, 0)),
                   pl.BlockSpec((rblk, D), lambda i: (i, 0))),
        out_shape=(jax.ShapeDtypeStruct((N, D), _f32),
                   jax.ShapeDtypeStruct((N, D), _f32)),
    )
    t0, t1 = tables(node_feat, We1[:D], We1[D:2 * D])

    mesh = plsc.VectorSubcoreMesh(
        core_axis_name="c", subcore_axis_name="s",
        num_cores=NC, num_subcores=NS)
    sc_params = pltpu.CompilerParams(needs_layout_passes=False)

    gather = pl.kernel(
        _sc_gather_body,
        out_type=(jax.ShapeDtypeStruct((E, D), _f32),
                  jax.ShapeDtypeStruct((E, D), _f32),
                  jax.ShapeDtypeStruct((8, E), _f32)),
        mesh=mesh,
        scratch_types=[
            pltpu.VMEM((CHUNK,), _i32),
            pltpu.VMEM((CHUNK,), _i32),
            pltpu.VMEM((CHUNK, D), _f32),
            pltpu.VMEM((CHUNK, D), _f32),
            pltpu.VMEM((8, CHUNK), _f32),
            pltpu.VMEM((N // L, 8 * L), _f32),
            pltpu.SemaphoreType.DMA,
        ],
        compiler_params=sc_params,
    )
    g0, g1, cdt = gather(t0, t1, ei0, ei1, cp625)

    eblk = 2560
    edge_mlp = pl.pallas_call(
        _edge_body,
        grid=(E // eblk,),
        in_specs=[
            pl.BlockSpec((eblk, D), lambda i: (i, 0)),
            pl.BlockSpec((eblk, D), lambda i: (i, 0)),
            pl.BlockSpec((8, eblk), lambda i: (0, i)),
            pl.BlockSpec((1, D), lambda i: (0, 0)),
            pl.BlockSpec((1, D), lambda i: (0, 0)),
            pl.BlockSpec((D, D), lambda i: (0, 0)),
            pl.BlockSpec((1, D), lambda i: (0, 0)),
            pl.BlockSpec((D, D), lambda i: (0, 0)),
            pl.BlockSpec((1, D), lambda i: (0, 0)),
            pl.BlockSpec((1, D), lambda i: (0, 0)),
        ],
        out_specs=(pl.BlockSpec((eblk, D), lambda i: (i, 0)),
                   pl.BlockSpec((8, eblk), lambda i: (0, i))),
        out_shape=(jax.ShapeDtypeStruct((E, D), _f32),
                   jax.ShapeDtypeStruct((8, E), _f32)),
    )
    ef, tt = edge_mlp(g0, g1, cdt, We1[2 * D:2 * D + 1], be1[None], We2,
                      be2[None], Wc1, bc1[None], Wc2.T)

    scatter = pl.kernel(
        _sc_scatter_body,
        out_type=(jax.ShapeDtypeStruct((NC, NP, D), _f32),
                  jax.ShapeDtypeStruct((NC, 4 * NP), _f32)),
        mesh=mesh,
        scratch_types=[
            pltpu.VMEM((CHUNK,), _i32),
            pltpu.VMEM((4, CHUNK), _i32),
            pltpu.VMEM((CHUNK, D), _f32),
            pltpu.VMEM((8, CHUNK), _f32),
            pltpu.VMEM((BR, D), _f32),
            pltpu.VMEM((TPS,), _f32),
            pltpu.VMEM_SHARED((N, D), _f32),
            pltpu.VMEM_SHARED((4 * NP,), _f32),
        ],
        compiler_params=sc_params,
    )
    accd, t_out = scatter(ef, tt, ei0, jnp.zeros((BR, D), _f32),
                          jnp.zeros((TPS,), _f32))
    tx = jnp.reshape(t_out, (8, NP))

    nblk = 1280
    node_mlp = pl.pallas_call(
        _node_body,
        grid=(NP // nblk,),
        in_specs=[
            pl.BlockSpec((nblk, D), lambda i: (i, 0)),
            pl.BlockSpec((8, nblk), lambda i: (0, i)),
            pl.BlockSpec((NC, nblk, D), lambda i: (0, i, 0)),
            pl.BlockSpec((8, nblk), lambda i: (0, i)),
            pl.BlockSpec((D, D), lambda i: (0, 0)),
            pl.BlockSpec((D, D), lambda i: (0, 0)),
            pl.BlockSpec((1, D), lambda i: (0, 0)),
            pl.BlockSpec((D, D), lambda i: (0, 0)),
            pl.BlockSpec((1, D), lambda i: (0, 0)),
        ],
        out_specs=(pl.BlockSpec((nblk, D), lambda i: (i, 0)),
                   pl.BlockSpec((8, nblk), lambda i: (0, i))),
        out_shape=(jax.ShapeDtypeStruct((NP, D), _f32),
                   jax.ShapeDtypeStruct((8, NP), _f32)),
    )
    nout, coutT = node_mlp(nfp, cpT, accd, tx, Wn1[:D], Wn1[D:],
                           bn1[None], Wn2, bn2[None])
    return nout[:N], coutT[:CD, :N].T


# speedup vs baseline: 7.0829x; 1.2951x over previous
import jax
import jax.numpy as jnp
from jax import lax
from jax.experimental import pallas as pl
from jax.experimental.pallas import tpu as pltpu
from jax.experimental.pallas import tpu_sc as plsc

N = 10000
E = 320000
D = 128
CD = 3
NC = 2
NS = 16
NW = NC * NS
L = 16
CHUNK = 128
NCHUNK = E // CHUNK
CPW = (NCHUNK + NW - 1) // NW
BR = 40
NB = N // BR
BPS = (NB + NS - 1) // NS
NP = 10240
TPS = 4 * NP // NS

_f32 = jnp.float32
_i32 = jnp.int32


def _silu(x):
    return x * jax.nn.sigmoid(x)


def _tables_body(nf, wa, wb, t0, t1):
    x = nf[...]
    t0[...] = jnp.dot(x, wa[...], preferred_element_type=_f32)
    t1[...] = jnp.dot(x, wb[...], preferred_element_type=_f32)


def _edge_body(g0, g1, cdt, wlast, be1, we2, be2, wc1, bc1, wc2r, ef, tt):
    dT = cdt[...][0:CD, :]
    rad = lax.dot_general(dT * dT, jnp.ones((CD, 1), _f32),
                          (((0,), (0,)), ((), ())),
                          preferred_element_type=_f32)
    h = _silu(g0[...] + g1[...] + rad * wlast[...] + be1[...])
    e = _silu(jnp.dot(h, we2[...], preferred_element_type=_f32) + be2[...])
    c = _silu(jnp.dot(e, wc1[...], preferred_element_type=_f32) + bc1[...])
    sT = lax.dot_general(wc2r[...], c, (((1,), (1,)), ((), ())),
                         preferred_element_type=_f32)
    ef[...] = e
    nb = dT.shape[1]
    tt[...] = jnp.concatenate(
        [dT * sT, jnp.ones((1, nb), _f32), jnp.zeros((4, nb), _f32)], axis=0)


def _node_body(nf, cpT, acc, tx, wn1a, wn1b, bn1, wn2, bn2, nout, coutT):
    ad = acc[...]
    agg = ad[0] + ad[1]
    t = tx[...]
    t4 = t[0:4] + t[4:8]
    cnt = jnp.maximum(t4[CD:CD + 1, :], 1.0)
    caggT = t4[0:CD, :] / cnt
    coutT[...] = cpT[...] + jnp.concatenate(
        [caggT, jnp.zeros((8 - CD, t.shape[1]), _f32)], axis=0)
    x = nf[...]
    nh = _silu(jnp.dot(x, wn1a[...], preferred_element_type=_f32)
               + jnp.dot(agg, wn1b[...], preferred_element_type=_f32)
               + bn1[...])
    nout[...] = x + jnp.dot(nh, wn2[...], preferred_element_type=_f32) + bn2[...]


def _sc_gather_body(t0, t1, i0, i1, cp, g0, g1, cdt,
                    idx0_v, idx1_v, rows0_v, rows1_v, douts_v, coordv,
                    sem_i0, sem_i1, sem_g0, sem_g1, sem_w):
    cid = lax.axis_index("c")
    sid = lax.axis_index("s")
    w = sid * NC + cid
    sems_i = (sem_i0, sem_i1)
    sems_g = (sem_g0, sem_g1)
    pltpu.sync_copy(cp, coordv)

    def chunk_of(jj, s):
        return w + (2 * jj + s) * NW

    def drain_wb(c, s):
        @pl.when(c < NCHUNK)
        def _():
            pltpu.make_async_copy(
                rows0_v.at[s], g0.at[pl.ds(0, CHUNK)], sem_w).wait()
            pltpu.make_async_copy(
                rows1_v.at[s], g1.at[pl.ds(0, CHUNK)], sem_w).wait()
            pltpu.make_async_copy(
                douts_v.at[s], cdt.at[:, pl.ds(0, CHUNK)], sem_w).wait()

    def body(jj, carry):
        for s in range(2):
            drain_wb(jnp.where(jj > 0, chunk_of(jj - 1, s), NCHUNK), s)
        for s in range(2):
            c = chunk_of(jj, s)

            @pl.when(c < NCHUNK)
            def _(s=s, c=c):
                base = c * CHUNK
                pltpu.async_copy(i0.at[pl.ds(base, CHUNK)], idx0_v.at[s],
                                 sems_i[s])
                pltpu.async_copy(i1.at[pl.ds(base, CHUNK)], idx1_v.at[s],
                                 sems_i[s])
        for s in range(2):
            c = chunk_of(jj, s)

            @pl.when(c < NCHUNK)
            def _(s=s, c=c):
                pltpu.make_async_copy(
                    i0.at[pl.ds(0, CHUNK)], idx0_v.at[s], sems_i[s]).wait()
                pltpu.make_async_copy(
                    i1.at[pl.ds(0, CHUNK)], idx1_v.at[s], sems_i[s]).wait()
                pltpu.async_copy(t0.at[idx0_v.at[s]], rows0_v.at[s], sems_g[s])
                pltpu.async_copy(t1.at[idx1_v.at[s]], rows1_v.at[s], sems_g[s])
        for s in range(2):
            c = chunk_of(jj, s)

            @pl.when(c < NCHUNK)
            def _(s=s):
                for g in range(CHUNK // L):
                    ids0 = idx0_v[s, pl.ds(g * L, L)]
                    ids1 = idx1_v[s, pl.ds(g * L, L)]
                    p0 = ids0 * 3
                    p1 = ids1 * 3
                    for d in range(CD):
                        c0 = plsc.load_gather(coordv, [p0 + d])
                        c1 = plsc.load_gather(coordv, [p1 + d])
                        douts_v[s, d, pl.ds(g * L, L)] = c0 - c1
        for s in range(2):
            c = chunk_of(jj, s)

            @pl.when(c < NCHUNK)
            def _(s=s, c=c):
                base = c * CHUNK
                pltpu.make_async_copy(
                    t0.at[pl.ds(0, CHUNK)], rows0_v.at[s], sems_g[s]).wait()
                pltpu.make_async_copy(
                    t1.at[pl.ds(0, CHUNK)], rows1_v.at[s], sems_g[s]).wait()
                pltpu.async_copy(rows0_v.at[s], g0.at[pl.ds(base, CHUNK)],
                                 sem_w)
                pltpu.async_copy(rows1_v.at[s], g1.at[pl.ds(base, CHUNK)],
                                 sem_w)
                pltpu.async_copy(douts_v.at[s], cdt.at[:, pl.ds(base, CHUNK)],
                                 sem_w)
        return carry

    npair = (CPW + 1) // 2
    lax.fori_loop(0, npair, body, 0)
    for s in range(2):
        drain_wb(chunk_of(npair - 1, s), s)


def _sc_scatter_body(ef, tt, i0, z128, z1, acc_out, t_out,
                     idx_v, idx4_v, rows_v, ttv, zbuf_v, zb1_v, acc_sh,
                     acct_sh, sem_in0, sem_in1, sem_s0, sem_s1):
    cid = lax.axis_index("c")
    sid = lax.axis_index("s")
    w = sid * NC + cid

    pltpu.sync_copy(z128, zbuf_v)
    pltpu.sync_copy(z1, zb1_v)
    for j in range(BPS):
        b = sid + j * NS

        @pl.when(b < NB)
        def _():
            pltpu.sync_copy(zbuf_v, acc_sh.at[pl.ds(b * BR, BR)])

    pltpu.sync_copy(zb1_v, acct_sh.at[pl.ds(sid * TPS, TPS)])
    plsc.subcore_barrier()

    sems_in = (sem_in0, sem_in1)
    sems_s = (sem_s0, sem_s1)

    def chunk_of(jj, s):
        return w + (2 * jj + s) * NW

    def drain_scat(c, s):
        @pl.when(c < NCHUNK)
        def _():
            pltpu.make_async_copy(
                rows_v.at[s], acc_sh.at[idx_v.at[s]], sems_s[s]).wait()
            for d in range(4):
                pltpu.make_async_copy(
                    ttv.at[s, d], acct_sh.at[idx4_v.at[s, d]],
                    sems_s[s]).wait()

    def body(jj, carry):
        for s in range(2):
            drain_scat(jnp.where(jj > 0, chunk_of(jj - 1, s), NCHUNK), s)
        for s in range(2):
            c = chunk_of(jj, s)

            @pl.when(c < NCHUNK)
            def _(s=s, c=c):
                base = c * CHUNK
                pltpu.async_copy(i0.at[pl.ds(base, CHUNK)], idx_v.at[s],
                                 sems_in[s])
                pltpu.async_copy(ef.at[pl.ds(base, CHUNK)], rows_v.at[s],
                                 sems_in[s])
                pltpu.async_copy(tt.at[:, pl.ds(base, CHUNK)], ttv.at[s],
                                 sems_in[s])
        for s in range(2):
            c = chunk_of(jj, s)

            @pl.when(c < NCHUNK)
            def _(s=s, c=c):
                pltpu.make_async_copy(
                    i0.at[pl.ds(0, CHUNK)], idx_v.at[s], sems_in[s]).wait()
                pltpu.make_async_copy(
                    ef.at[pl.ds(0, CHUNK)], rows_v.at[s], sems_in[s]).wait()
                pltpu.make_async_copy(
                    tt.at[:, pl.ds(0, CHUNK)], ttv.at[s], sems_in[s]).wait()
                pltpu.async_copy(rows_v.at[s], acc_sh.at[idx_v.at[s]],
                                 sems_s[s], add=True)
                for g in range(CHUNK // L):
                    ids = idx_v[s, pl.ds(g * L, L)]
                    for d in range(4):
                        idx4_v[s, d, pl.ds(g * L, L)] = ids + d * NP
                for d in range(4):
                    pltpu.async_copy(ttv.at[s, d],
                                     acct_sh.at[idx4_v.at[s, d]],
                                     sems_s[s], add=True)
        return carry

    npair = (CPW + 1) // 2
    lax.fori_loop(0, npair, body, 0)
    for s in range(2):
        drain_scat(chunk_of(npair - 1, s), s)
    plsc.subcore_barrier()

    for j in range(BPS):
        b = sid + j * NS

        @pl.when(b < NB)
        def _():
            pltpu.sync_copy(acc_sh.at[pl.ds(b * BR, BR)], zbuf_v)
            pltpu.sync_copy(zbuf_v, acc_out.at[cid, pl.ds(b * BR, BR), :])

    pltpu.sync_copy(acct_sh.at[pl.ds(sid * TPS, TPS)], zb1_v)
    pltpu.sync_copy(zb1_v, t_out.at[cid, pl.ds(sid * TPS, TPS)])


def kernel(node_feat, coord, edge_index, We1, be1, We2, be2,
           Wn1, bn1, Wn2, bn2, Wc1, bc1, Wc2):
    ei0 = edge_index[0]
    ei1 = edge_index[1]
    coord8 = jnp.pad(coord, ((0, 0), (0, 8 - CD)))
    cp1 = jnp.reshape(coord, (CD * N,))
    cpT = jnp.pad(coord8, ((0, NP - N), (0, 0))).T
    nfp = jnp.pad(node_feat, ((0, NP - N), (0, 0)))

    rblk = 2000
    tables = pl.pallas_call(
        _tables_body,
        grid=(N // rblk,),
        in_specs=[
            pl.BlockSpec((rblk, D), lambda i: (i, 0)),
            pl.BlockSpec((D, D), lambda i: (0, 0)),
            pl.BlockSpec((D, D), lambda i: (0, 0)),
        ],
        out_specs=(pl.BlockSpec((rblk, D), lambda i: (i, 0)),
                   pl.BlockSpec((rblk, D), lambda i: (i, 0))),
        out_shape=(jax.ShapeDtypeStruct((N, D), _f32),
                   jax.ShapeDtypeStruct((N, D), _f32)),
    )
    t0, t1 = tables(node_feat, We1[:D], We1[D:2 * D])

    mesh = plsc.VectorSubcoreMesh(
        core_axis_name="c", subcore_axis_name="s",
        num_cores=NC, num_subcores=NS)
    sc_params = pltpu.CompilerParams(needs_layout_passes=False)

    gather = pl.kernel(
        _sc_gather_body,
        out_type=(jax.ShapeDtypeStruct((E, D), _f32),
                  jax.ShapeDtypeStruct((E, D), _f32),
                  jax.ShapeDtypeStruct((8, E), _f32)),
        mesh=mesh,
        scratch_types=[
            pltpu.VMEM((2, CHUNK), _i32),
            pltpu.VMEM((2, CHUNK), _i32),
            pltpu.VMEM((2, CHUNK, D), _f32),
            pltpu.VMEM((2, CHUNK, D), _f32),
            pltpu.VMEM((2, 8, CHUNK), _f32),
            pltpu.VMEM((CD * N,), _f32),
            pltpu.SemaphoreType.DMA,
            pltpu.SemaphoreType.DMA,
            pltpu.SemaphoreType.DMA,
            pltpu.SemaphoreType.DMA,
            pltpu.SemaphoreType.DMA,
        ],
        compiler_params=sc_params,
    )
    g0, g1, cdt = gather(t0, t1, ei0, ei1, cp1)

    eblk = 2560
    edge_mlp = pl.pallas_call(
        _edge_body,
        grid=(E // eblk,),
        in_specs=[
            pl.BlockSpec((eblk, D), lambda i: (i, 0)),
            pl.BlockSpec((eblk, D), lambda i: (i, 0)),
            pl.BlockSpec((8, eblk), lambda i: (0, i)),
            pl.BlockSpec((1, D), lambda i: (0, 0)),
            pl.BlockSpec((1, D), lambda i: (0, 0)),
            pl.BlockSpec((D, D), lambda i: (0, 0)),
            pl.BlockSpec((1, D), lambda i: (0, 0)),
            pl.BlockSpec((D, D), lambda i: (0, 0)),
            pl.BlockSpec((1, D), lambda i: (0, 0)),
            pl.BlockSpec((1, D), lambda i: (0, 0)),
        ],
        out_specs=(pl.BlockSpec((eblk, D), lambda i: (i, 0)),
                   pl.BlockSpec((8, eblk), lambda i: (0, i))),
        out_shape=(jax.ShapeDtypeStruct((E, D), _f32),
                   jax.ShapeDtypeStruct((8, E), _f32)),
    )
    ef, tt = edge_mlp(g0, g1, cdt, We1[2 * D:2 * D + 1], be1[None], We2,
                      be2[None], Wc1, bc1[None], Wc2.T)

    scatter = pl.kernel(
        _sc_scatter_body,
        out_type=(jax.ShapeDtypeStruct((NC, NP, D), _f32),
                  jax.ShapeDtypeStruct((NC, 4 * NP), _f32)),
        mesh=mesh,
        scratch_types=[
            pltpu.VMEM((2, CHUNK), _i32),
            pltpu.VMEM((2, 4, CHUNK), _i32),
            pltpu.VMEM((2, CHUNK, D), _f32),
            pltpu.VMEM((2, 8, CHUNK), _f32),
            pltpu.VMEM((BR, D), _f32),
            pltpu.VMEM((TPS,), _f32),
            pltpu.VMEM_SHARED((N, D), _f32),
            pltpu.VMEM_SHARED((4 * NP,), _f32),
            pltpu.SemaphoreType.DMA,
            pltpu.SemaphoreType.DMA,
            pltpu.SemaphoreType.DMA,
            pltpu.SemaphoreType.DMA,
        ],
        compiler_params=sc_params,
    )
    accd, t_out = scatter(ef, tt, ei0, jnp.zeros((BR, D), _f32),
                          jnp.zeros((TPS,), _f32))
    tx = jnp.reshape(t_out, (8, NP))

    nblk = 1280
    node_mlp = pl.pallas_call(
        _node_body,
        grid=(NP // nblk,),
        in_specs=[
            pl.BlockSpec((nblk, D), lambda i: (i, 0)),
            pl.BlockSpec((8, nblk), lambda i: (0, i)),
            pl.BlockSpec((NC, nblk, D), lambda i: (0, i, 0)),
            pl.BlockSpec((8, nblk), lambda i: (0, i)),
            pl.BlockSpec((D, D), lambda i: (0, 0)),
            pl.BlockSpec((D, D), lambda i: (0, 0)),
            pl.BlockSpec((1, D), lambda i: (0, 0)),
            pl.BlockSpec((D, D), lambda i: (0, 0)),
            pl.BlockSpec((1, D), lambda i: (0, 0)),
        ],
        out_specs=(pl.BlockSpec((nblk, D), lambda i: (i, 0)),
                   pl.BlockSpec((8, nblk), lambda i: (0, i))),
        out_shape=(jax.ShapeDtypeStruct((NP, D), _f32),
                   jax.ShapeDtypeStruct((8, NP), _f32)),
    )
    nout, coutT = node_mlp(nfp, cpT, accd, tx, Wn1[:D], Wn1[D:],
                           bn1[None], Wn2, bn2[None])
    return nout[:N], coutT[:CD, :N].T


# speedup vs baseline: 7.6203x; 1.0759x over previous
import jax
import jax.numpy as jnp
from jax import lax
from jax.experimental import pallas as pl
from jax.experimental.pallas import tpu as pltpu
from jax.experimental.pallas import tpu_sc as plsc

N = 10000
E = 320000
D = 128
CD = 3
NC = 2
NS = 16
NW = NC * NS
L = 16
CHUNK = 128
NCHUNK = E // CHUNK
CPW = (NCHUNK + NW - 1) // NW
BR = 40
NB = N // BR
BPS = (NB + NS - 1) // NS
NP = 10240
TPS = 4 * NP // NS

_f32 = jnp.float32
_i32 = jnp.int32


def _silu(x):
    return x * jax.nn.sigmoid(x)


def _tables_body(nf, wa, wb, t0, t1):
    x = nf[...]
    t0[...] = jnp.dot(x, wa[...], preferred_element_type=_f32)
    t1[...] = jnp.dot(x, wb[...], preferred_element_type=_f32)


def _edge_body(g0, cdt, wlast, be1, we2, be2, wc1, bc1, wc2r, ef, tt):
    dT = cdt[...][0:CD, :]
    rad = lax.dot_general(dT * dT, jnp.ones((CD, 1), _f32),
                          (((0,), (0,)), ((), ())),
                          preferred_element_type=_f32)
    h = _silu(g0[...] + rad * wlast[...] + be1[...])
    e = _silu(jnp.dot(h, we2[...], preferred_element_type=_f32) + be2[...])
    c = _silu(jnp.dot(e, wc1[...], preferred_element_type=_f32) + bc1[...])
    sT = lax.dot_general(wc2r[...], c, (((1,), (1,)), ((), ())),
                         preferred_element_type=_f32)
    ef[...] = e
    nb = dT.shape[1]
    tt[...] = jnp.concatenate(
        [dT * sT, jnp.ones((1, nb), _f32), jnp.zeros((4, nb), _f32)], axis=0)


def _node_body(nf, cpT, acc, tx, wn1a, wn1b, bn1, wn2, bn2, nout, coutT):
    ad = acc[...]
    agg = ad[0] + ad[1]
    t = tx[...]
    t4 = t[0:4] + t[4:8]
    cnt = jnp.maximum(t4[CD:CD + 1, :], 1.0)
    caggT = t4[0:CD, :] / cnt
    coutT[...] = cpT[...] + jnp.concatenate(
        [caggT, jnp.zeros((8 - CD, t.shape[1]), _f32)], axis=0)
    x = nf[...]
    nh = _silu(jnp.dot(x, wn1a[...], preferred_element_type=_f32)
               + jnp.dot(agg, wn1b[...], preferred_element_type=_f32)
               + bn1[...])
    nout[...] = x + jnp.dot(nh, wn2[...], preferred_element_type=_f32) + bn2[...]


def _sc_gather_body(t0, t1, i0, i1, cp, g0, cdt,
                    idx0_v, idx1_v, rows_v, douts_v, coordv,
                    sem_i0, sem_i1, sem_g0, sem_g1, sem_w):
    cid = lax.axis_index("c")
    sid = lax.axis_index("s")
    w = sid * NC + cid
    sems_i = (sem_i0, sem_i1)
    sems_g = (sem_g0, sem_g1)
    pltpu.sync_copy(cp, coordv)

    def chunk_of(jj, s):
        return w + (2 * jj + s) * NW

    def drain_wb(c, s):
        @pl.when(c < NCHUNK)
        def _():
            pltpu.make_async_copy(
                rows_v.at[s], g0.at[pl.ds(0, CHUNK)], sem_w).wait()
            pltpu.make_async_copy(
                douts_v.at[s], cdt.at[:, pl.ds(0, CHUNK)], sem_w).wait()

    def body(jj, carry):
        for s in range(2):
            drain_wb(jnp.where(jj > 0, chunk_of(jj - 1, s), NCHUNK), s)
        for s in range(2):
            c = chunk_of(jj, s)

            @pl.when(c < NCHUNK)
            def _(s=s, c=c):
                base = c * CHUNK
                pltpu.async_copy(i0.at[pl.ds(base, CHUNK)], idx0_v.at[s],
                                 sems_i[s])
                pltpu.async_copy(i1.at[pl.ds(base, CHUNK)], idx1_v.at[s],
                                 sems_i[s])
        for s in range(2):
            c = chunk_of(jj, s)

            @pl.when(c < NCHUNK)
            def _(s=s, c=c):
                pltpu.make_async_copy(
                    i0.at[pl.ds(0, CHUNK)], idx0_v.at[s], sems_i[s]).wait()
                pltpu.make_async_copy(
                    i1.at[pl.ds(0, CHUNK)], idx1_v.at[s], sems_i[s]).wait()
                pltpu.async_copy(t0.at[idx0_v.at[s]], rows_v.at[s], sems_g[s])
        for s in range(2):
            c = chunk_of(jj, s)

            @pl.when(c < NCHUNK)
            def _(s=s):
                for g in range(CHUNK // L):
                    ids0 = idx0_v[s, pl.ds(g * L, L)]
                    ids1 = idx1_v[s, pl.ds(g * L, L)]
                    p0 = ids0 * 3
                    p1 = ids1 * 3
                    for d in range(CD):
                        c0 = plsc.load_gather(coordv, [p0 + d])
                        c1 = plsc.load_gather(coordv, [p1 + d])
                        douts_v[s, d, pl.ds(g * L, L)] = c0 - c1
        for s in range(2):
            c = chunk_of(jj, s)

            @pl.when(c < NCHUNK)
            def _(s=s, c=c):
                pltpu.make_async_copy(
                    t0.at[pl.ds(0, CHUNK)], rows_v.at[s], sems_g[s]).wait()
                pltpu.async_copy(t1.at[idx1_v.at[s]], rows_v.at[s],
                                 sems_g[s], add=True)
        for s in range(2):
            c = chunk_of(jj, s)

            @pl.when(c < NCHUNK)
            def _(s=s, c=c):
                base = c * CHUNK
                pltpu.make_async_copy(
                    t1.at[pl.ds(0, CHUNK)], rows_v.at[s], sems_g[s]).wait()
                pltpu.async_copy(rows_v.at[s], g0.at[pl.ds(base, CHUNK)],
                                 sem_w)
                pltpu.async_copy(douts_v.at[s], cdt.at[:, pl.ds(base, CHUNK)],
                                 sem_w)
        return carry

    npair = (CPW + 1) // 2
    lax.fori_loop(0, npair, body, 0)
    for s in range(2):
        drain_wb(chunk_of(npair - 1, s), s)


def _sc_scatter_body(ef, tt, i0, z128, z1, acc_out, t_out,
                     idx_v, idx4_v, rows_v, ttv, zbuf_v, zb1_v, acc_sh,
                     acct_sh, sem_in0, sem_in1, sem_s0, sem_s1):
    cid = lax.axis_index("c")
    sid = lax.axis_index("s")
    w = sid * NC + cid

    pltpu.sync_copy(z128, zbuf_v)
    pltpu.sync_copy(z1, zb1_v)
    for j in range(BPS):
        b = sid + j * NS

        @pl.when(b < NB)
        def _():
            pltpu.sync_copy(zbuf_v, acc_sh.at[pl.ds(b * BR, BR)])

    pltpu.sync_copy(zb1_v, acct_sh.at[pl.ds(sid * TPS, TPS)])
    plsc.subcore_barrier()

    sems_in = (sem_in0, sem_in1)
    sems_s = (sem_s0, sem_s1)

    def chunk_of(jj, s):
        return w + (2 * jj + s) * NW

    def drain_scat(c, s):
        @pl.when(c < NCHUNK)
        def _():
            pltpu.make_async_copy(
                rows_v.at[s], acc_sh.at[idx_v.at[s]], sems_s[s]).wait()
            for d in range(4):
                pltpu.make_async_copy(
                    ttv.at[s, d], acct_sh.at[idx4_v.at[s, d]],
                    sems_s[s]).wait()

    def body(jj, carry):
        for s in range(2):
            drain_scat(jnp.where(jj > 0, chunk_of(jj - 1, s), NCHUNK), s)
        for s in range(2):
            c = chunk_of(jj, s)

            @pl.when(c < NCHUNK)
            def _(s=s, c=c):
                base = c * CHUNK
                pltpu.async_copy(i0.at[pl.ds(base, CHUNK)], idx_v.at[s],
                                 sems_in[s])
                pltpu.async_copy(ef.at[pl.ds(base, CHUNK)], rows_v.at[s],
                                 sems_in[s])
                pltpu.async_copy(tt.at[:, pl.ds(base, CHUNK)], ttv.at[s],
                                 sems_in[s])
        for s in range(2):
            c = chunk_of(jj, s)

            @pl.when(c < NCHUNK)
            def _(s=s, c=c):
                pltpu.make_async_copy(
                    i0.at[pl.ds(0, CHUNK)], idx_v.at[s], sems_in[s]).wait()
                pltpu.make_async_copy(
                    ef.at[pl.ds(0, CHUNK)], rows_v.at[s], sems_in[s]).wait()
                pltpu.make_async_copy(
                    tt.at[:, pl.ds(0, CHUNK)], ttv.at[s], sems_in[s]).wait()
                pltpu.async_copy(rows_v.at[s], acc_sh.at[idx_v.at[s]],
                                 sems_s[s], add=True)
                for g in range(CHUNK // L):
                    ids = idx_v[s, pl.ds(g * L, L)]
                    for d in range(4):
                        idx4_v[s, d, pl.ds(g * L, L)] = ids + d * NP
                for d in range(4):
                    pltpu.async_copy(ttv.at[s, d],
                                     acct_sh.at[idx4_v.at[s, d]],
                                     sems_s[s], add=True)
        return carry

    npair = (CPW + 1) // 2
    lax.fori_loop(0, npair, body, 0)
    for s in range(2):
        drain_scat(chunk_of(npair - 1, s), s)
    plsc.subcore_barrier()

    for j in range(BPS):
        b = sid + j * NS

        @pl.when(b < NB)
        def _():
            pltpu.sync_copy(acc_sh.at[pl.ds(b * BR, BR)], zbuf_v)
            pltpu.sync_copy(zbuf_v, acc_out.at[cid, pl.ds(b * BR, BR), :])

    pltpu.sync_copy(acct_sh.at[pl.ds(sid * TPS, TPS)], zb1_v)
    pltpu.sync_copy(zb1_v, t_out.at[cid, pl.ds(sid * TPS, TPS)])


def kernel(node_feat, coord, edge_index, We1, be1, We2, be2,
           Wn1, bn1, Wn2, bn2, Wc1, bc1, Wc2):
    ei0 = edge_index[0]
    ei1 = edge_index[1]
    coord8 = jnp.pad(coord, ((0, 0), (0, 8 - CD)))
    cp1 = jnp.reshape(coord, (CD * N,))
    cpT = jnp.pad(coord8, ((0, NP - N), (0, 0))).T
    nfp = jnp.pad(node_feat, ((0, NP - N), (0, 0)))

    rblk = 2000
    tables = pl.pallas_call(
        _tables_body,
        grid=(N // rblk,),
        in_specs=[
            pl.BlockSpec((rblk, D), lambda i: (i, 0)),
            pl.BlockSpec((D, D), lambda i: (0, 0)),
            pl.BlockSpec((D, D), lambda i: (0, 0)),
        ],
        out_specs=(pl.BlockSpec((rblk, D), lambda i: (i, 0)),
                   pl.BlockSpec((rblk, D), lambda i: (i, 0))),
        out_shape=(jax.ShapeDtypeStruct((N, D), _f32),
                   jax.ShapeDtypeStruct((N, D), _f32)),
    )
    t0, t1 = tables(node_feat, We1[:D], We1[D:2 * D])

    mesh = plsc.VectorSubcoreMesh(
        core_axis_name="c", subcore_axis_name="s",
        num_cores=NC, num_subcores=NS)
    sc_params = pltpu.CompilerParams(needs_layout_passes=False)

    gather = pl.kernel(
        _sc_gather_body,
        out_type=(jax.ShapeDtypeStruct((E, D), _f32),
                  jax.ShapeDtypeStruct((8, E), _f32)),
        mesh=mesh,
        scratch_types=[
            pltpu.VMEM((2, CHUNK), _i32),
            pltpu.VMEM((2, CHUNK), _i32),
            pltpu.VMEM((2, CHUNK, D), _f32),
            pltpu.VMEM((2, 8, CHUNK), _f32),
            pltpu.VMEM((CD * N,), _f32),
            pltpu.SemaphoreType.DMA,
            pltpu.SemaphoreType.DMA,
            pltpu.SemaphoreType.DMA,
            pltpu.SemaphoreType.DMA,
            pltpu.SemaphoreType.DMA,
        ],
        compiler_params=sc_params,
    )
    g0, cdt = gather(t0, t1, ei0, ei1, cp1)

    eblk = 2560
    edge_mlp = pl.pallas_call(
        _edge_body,
        grid=(E // eblk,),
        in_specs=[
            pl.BlockSpec((eblk, D), lambda i: (i, 0)),
            pl.BlockSpec((8, eblk), lambda i: (0, i)),
            pl.BlockSpec((1, D), lambda i: (0, 0)),
            pl.BlockSpec((1, D), lambda i: (0, 0)),
            pl.BlockSpec((D, D), lambda i: (0, 0)),
            pl.BlockSpec((1, D), lambda i: (0, 0)),
            pl.BlockSpec((D, D), lambda i: (0, 0)),
            pl.BlockSpec((1, D), lambda i: (0, 0)),
            pl.BlockSpec((1, D), lambda i: (0, 0)),
        ],
        out_specs=(pl.BlockSpec((eblk, D), lambda i: (i, 0)),
                   pl.BlockSpec((8, eblk), lambda i: (0, i))),
        out_shape=(jax.ShapeDtypeStruct((E, D), _f32),
                   jax.ShapeDtypeStruct((8, E), _f32)),
    )
    ef, tt = edge_mlp(g0, cdt, We1[2 * D:2 * D + 1], be1[None], We2,
                      be2[None], Wc1, bc1[None], Wc2.T)

    scatter = pl.kernel(
        _sc_scatter_body,
        out_type=(jax.ShapeDtypeStruct((NC, NP, D), _f32),
                  jax.ShapeDtypeStruct((NC, 4 * NP), _f32)),
        mesh=mesh,
        scratch_types=[
            pltpu.VMEM((2, CHUNK), _i32),
            pltpu.VMEM((2, 4, CHUNK), _i32),
            pltpu.VMEM((2, CHUNK, D), _f32),
            pltpu.VMEM((2, 8, CHUNK), _f32),
            pltpu.VMEM((BR, D), _f32),
            pltpu.VMEM((TPS,), _f32),
            pltpu.VMEM_SHARED((N, D), _f32),
            pltpu.VMEM_SHARED((4 * NP,), _f32),
            pltpu.SemaphoreType.DMA,
            pltpu.SemaphoreType.DMA,
            pltpu.SemaphoreType.DMA,
            pltpu.SemaphoreType.DMA,
        ],
        compiler_params=sc_params,
    )
    accd, t_out = scatter(ef, tt, ei0, jnp.zeros((BR, D), _f32),
                          jnp.zeros((TPS,), _f32))
    tx = jnp.reshape(t_out, (8, NP))

    nblk = 1280
    node_mlp = pl.pallas_call(
        _node_body,
        grid=(NP // nblk,),
        in_specs=[
            pl.BlockSpec((nblk, D), lambda i: (i, 0)),
            pl.BlockSpec((8, nblk), lambda i: (0, i)),
            pl.BlockSpec((NC, nblk, D), lambda i: (0, i, 0)),
            pl.BlockSpec((8, nblk), lambda i: (0, i)),
            pl.BlockSpec((D, D), lambda i: (0, 0)),
            pl.BlockSpec((D, D), lambda i: (0, 0)),
            pl.BlockSpec((1, D), lambda i: (0, 0)),
            pl.BlockSpec((D, D), lambda i: (0, 0)),
            pl.BlockSpec((1, D), lambda i: (0, 0)),
        ],
        out_specs=(pl.BlockSpec((nblk, D), lambda i: (i, 0)),
                   pl.BlockSpec((8, nblk), lambda i: (0, i))),
        out_shape=(jax.ShapeDtypeStruct((NP, D), _f32),
                   jax.ShapeDtypeStruct((8, NP), _f32)),
    )
    nout, coutT = node_mlp(nfp, cpT, accd, tx, Wn1[:D], Wn1[D:],
                           bn1[None], Wn2, bn2[None])
    return nout[:N], coutT[:CD, :N].T


# speedup vs baseline: 9.4924x; 1.2457x over previous
import jax
import jax.numpy as jnp
from jax import lax
from jax.experimental import pallas as pl
from jax.experimental.pallas import tpu as pltpu
from jax.experimental.pallas import tpu_sc as plsc

N = 10000
E = 320000
D = 128
CD = 3
NC = 2
NS = 16
NW = NC * NS
L = 16
CHUNK = 128
NSL = 2
ES = E // NSL
NCHUNK = ES // CHUNK
CPW = (NCHUNK + NW - 1) // NW
BR = 40
NB = N // BR
BPS = (NB + NS - 1) // NS
NP = 10240
TPS = 4 * NP // NS

_f32 = jnp.float32
_i32 = jnp.int32


def _silu(x):
    return x * jax.nn.sigmoid(x)


def _tables_body(nf, wa, wb, t0, t1):
    x = nf[...]
    t0[...] = jnp.dot(x, wa[...], preferred_element_type=_f32)
    t1[...] = jnp.dot(x, wb[...], preferred_element_type=_f32)


def _edge_body(g0, cdt, wlast, be1, we2, be2, wc1, bc1, wc2r, ef, tt):
    dT = cdt[...][0:CD, :]
    rad = lax.dot_general(dT * dT, jnp.ones((CD, 1), _f32),
                          (((0,), (0,)), ((), ())),
                          preferred_element_type=_f32)
    h = _silu(g0[...] + rad * wlast[...] + be1[...])
    e = _silu(jnp.dot(h, we2[...], preferred_element_type=_f32) + be2[...])
    c = _silu(jnp.dot(e, wc1[...], preferred_element_type=_f32) + bc1[...])
    sT = lax.dot_general(wc2r[...], c, (((1,), (1,)), ((), ())),
                         preferred_element_type=_f32)
    ef[...] = e
    nb = dT.shape[1]
    tt[...] = jnp.concatenate(
        [dT * sT, jnp.ones((1, nb), _f32), jnp.zeros((4, nb), _f32)], axis=0)


def _node_body(nf, cpT, acc, acc2, tx, tx2, wn1a, wn1b, bn1, wn2, bn2,
               nout, coutT):
    ad = acc[...]
    ad2 = acc2[...]
    agg = (ad[0] + ad[1]) + (ad2[0] + ad2[1])
    t = tx[...] + tx2[...]
    t4 = t[0:4] + t[4:8]
    cnt = jnp.maximum(t4[CD:CD + 1, :], 1.0)
    caggT = t4[0:CD, :] / cnt
    coutT[...] = cpT[...] + jnp.concatenate(
        [caggT, jnp.zeros((8 - CD, t.shape[1]), _f32)], axis=0)
    x = nf[...]
    nh = _silu(jnp.dot(x, wn1a[...], preferred_element_type=_f32)
               + jnp.dot(agg, wn1b[...], preferred_element_type=_f32)
               + bn1[...])
    nout[...] = x + jnp.dot(nh, wn2[...], preferred_element_type=_f32) + bn2[...]


def _sc_gather_body(t0, t1, i0, i1, cp, g0, cdt,
                    idx0_v, idx1_v, rows_v, douts_v, coordv,
                    sem_i0, sem_i1, sem_g0, sem_g1, sem_w):
    cid = lax.axis_index("c")
    sid = lax.axis_index("s")
    w = sid * NC + cid
    sems_i = (sem_i0, sem_i1)
    sems_g = (sem_g0, sem_g1)
    pltpu.sync_copy(cp, coordv)

    def chunk_of(jj, s):
        return w + (2 * jj + s) * NW

    def drain_wb(c, s):
        @pl.when(c < NCHUNK)
        def _():
            pltpu.make_async_copy(
                rows_v.at[s], g0.at[pl.ds(0, CHUNK)], sem_w).wait()
            pltpu.make_async_copy(
                douts_v.at[s], cdt.at[:, pl.ds(0, CHUNK)], sem_w).wait()

    def body(jj, carry):
        for s in range(2):
            drain_wb(jnp.where(jj > 0, chunk_of(jj - 1, s), NCHUNK), s)
        for s in range(2):
            c = chunk_of(jj, s)

            @pl.when(c < NCHUNK)
            def _(s=s, c=c):
                base = c * CHUNK
                pltpu.async_copy(i0.at[pl.ds(base, CHUNK)], idx0_v.at[s],
                                 sems_i[s])
                pltpu.async_copy(i1.at[pl.ds(base, CHUNK)], idx1_v.at[s],
                                 sems_i[s])
        for s in range(2):
            c = chunk_of(jj, s)

            @pl.when(c < NCHUNK)
            def _(s=s, c=c):
                pltpu.make_async_copy(
                    i0.at[pl.ds(0, CHUNK)], idx0_v.at[s], sems_i[s]).wait()
                pltpu.make_async_copy(
                    i1.at[pl.ds(0, CHUNK)], idx1_v.at[s], sems_i[s]).wait()
                pltpu.async_copy(t0.at[idx0_v.at[s]], rows_v.at[s], sems_g[s])
        for s in range(2):
            c = chunk_of(jj, s)

            @pl.when(c < NCHUNK)
            def _(s=s):
                for g in range(CHUNK // L):
                    ids0 = idx0_v[s, pl.ds(g * L, L)]
                    ids1 = idx1_v[s, pl.ds(g * L, L)]
                    p0 = ids0 * 3
                    p1 = ids1 * 3
                    for d in range(CD):
                        c0 = plsc.load_gather(coordv, [p0 + d])
                        c1 = plsc.load_gather(coordv, [p1 + d])
                        douts_v[s, d, pl.ds(g * L, L)] = c0 - c1
        for s in range(2):
            c = chunk_of(jj, s)

            @pl.when(c < NCHUNK)
            def _(s=s, c=c):
                pltpu.make_async_copy(
                    t0.at[pl.ds(0, CHUNK)], rows_v.at[s], sems_g[s]).wait()
                pltpu.async_copy(t1.at[idx1_v.at[s]], rows_v.at[s],
                                 sems_g[s], add=True)
        for s in range(2):
            c = chunk_of(jj, s)

            @pl.when(c < NCHUNK)
            def _(s=s, c=c):
                base = c * CHUNK
                pltpu.make_async_copy(
                    t1.at[pl.ds(0, CHUNK)], rows_v.at[s], sems_g[s]).wait()
                pltpu.async_copy(rows_v.at[s], g0.at[pl.ds(base, CHUNK)],
                                 sem_w)
                pltpu.async_copy(douts_v.at[s], cdt.at[:, pl.ds(base, CHUNK)],
                                 sem_w)
        return carry

    npair = (CPW + 1) // 2
    lax.fori_loop(0, npair, body, 0)
    for s in range(2):
        drain_wb(chunk_of(npair - 1, s), s)


def _sc_scatter_body(ef, tt, i0, z128, z1, acc_out, t_out,
                     idx_v, idx4_v, rows_v, ttv, zbuf_v, zb1_v, acc_sh,
                     acct_sh, sem_in0, sem_in1, sem_s0, sem_s1):
    cid = lax.axis_index("c")
    sid = lax.axis_index("s")
    w = sid * NC + cid

    pltpu.sync_copy(z128, zbuf_v)
    pltpu.sync_copy(z1, zb1_v)
    for j in range(BPS):
        b = sid + j * NS

        @pl.when(b < NB)
        def _():
            pltpu.sync_copy(zbuf_v, acc_sh.at[pl.ds(b * BR, BR)])

    pltpu.sync_copy(zb1_v, acct_sh.at[pl.ds(sid * TPS, TPS)])
    plsc.subcore_barrier()

    sems_in = (sem_in0, sem_in1)
    sems_s = (sem_s0, sem_s1)

    def chunk_of(jj, s):
        return w + (2 * jj + s) * NW

    def drain_scat(c, s):
        @pl.when(c < NCHUNK)
        def _():
            pltpu.make_async_copy(
                rows_v.at[s], acc_sh.at[idx_v.at[s]], sems_s[s]).wait()
            for d in range(4):
                pltpu.make_async_copy(
                    ttv.at[s, d], acct_sh.at[idx4_v.at[s, d]],
                    sems_s[s]).wait()

    def body(jj, carry):
        for s in range(2):
            drain_scat(jnp.where(jj > 0, chunk_of(jj - 1, s), NCHUNK), s)
        for s in range(2):
            c = chunk_of(jj, s)

            @pl.when(c < NCHUNK)
            def _(s=s, c=c):
                base = c * CHUNK
                pltpu.async_copy(i0.at[pl.ds(base, CHUNK)], idx_v.at[s],
                                 sems_in[s])
                pltpu.async_copy(ef.at[pl.ds(base, CHUNK)], rows_v.at[s],
                                 sems_in[s])
                pltpu.async_copy(tt.at[:, pl.ds(base, CHUNK)], ttv.at[s],
                                 sems_in[s])
        for s in range(2):
            c = chunk_of(jj, s)

            @pl.when(c < NCHUNK)
            def _(s=s, c=c):
                pltpu.make_async_copy(
                    i0.at[pl.ds(0, CHUNK)], idx_v.at[s], sems_in[s]).wait()
                pltpu.make_async_copy(
                    ef.at[pl.ds(0, CHUNK)], rows_v.at[s], sems_in[s]).wait()
                pltpu.make_async_copy(
                    tt.at[:, pl.ds(0, CHUNK)], ttv.at[s], sems_in[s]).wait()
                pltpu.async_copy(rows_v.at[s], acc_sh.at[idx_v.at[s]],
                                 sems_s[s], add=True)
                for g in range(CHUNK // L):
                    ids = idx_v[s, pl.ds(g * L, L)]
                    for d in range(4):
                        idx4_v[s, d, pl.ds(g * L, L)] = ids + d * NP
                for d in range(4):
                    pltpu.async_copy(ttv.at[s, d],
                                     acct_sh.at[idx4_v.at[s, d]],
                                     sems_s[s], add=True)
        return carry

    npair = (CPW + 1) // 2
    lax.fori_loop(0, npair, body, 0)
    for s in range(2):
        drain_scat(chunk_of(npair - 1, s), s)
    plsc.subcore_barrier()

    for j in range(BPS):
        b = sid + j * NS

        @pl.when(b < NB)
        def _():
            pltpu.sync_copy(acc_sh.at[pl.ds(b * BR, BR)], zbuf_v)
            pltpu.sync_copy(zbuf_v, acc_out.at[cid, pl.ds(b * BR, BR), :])

    pltpu.sync_copy(acct_sh.at[pl.ds(sid * TPS, TPS)], zb1_v)
    pltpu.sync_copy(zb1_v, t_out.at[cid, pl.ds(sid * TPS, TPS)])


def kernel(node_feat, coord, edge_index, We1, be1, We2, be2,
           Wn1, bn1, Wn2, bn2, Wc1, bc1, Wc2):
    ei0 = edge_index[0]
    ei1 = edge_index[1]
    coord8 = jnp.pad(coord, ((0, 0), (0, 8 - CD)))
    cp1 = jnp.reshape(coord, (CD * N,))
    cpT = jnp.pad(coord8, ((0, NP - N), (0, 0))).T
    nfp = jnp.pad(node_feat, ((0, NP - N), (0, 0)))

    rblk = 2000
    tables = pl.pallas_call(
        _tables_body,
        grid=(N // rblk,),
        in_specs=[
            pl.BlockSpec((rblk, D), lambda i: (i, 0)),
            pl.BlockSpec((D, D), lambda i: (0, 0)),
            pl.BlockSpec((D, D), lambda i: (0, 0)),
        ],
        out_specs=(pl.BlockSpec((rblk, D), lambda i: (i, 0)),
                   pl.BlockSpec((rblk, D), lambda i: (i, 0))),
        out_shape=(jax.ShapeDtypeStruct((N, D), _f32),
                   jax.ShapeDtypeStruct((N, D), _f32)),
    )
    t0, t1 = tables(node_feat, We1[:D], We1[D:2 * D])

    mesh = plsc.VectorSubcoreMesh(
        core_axis_name="c", subcore_axis_name="s",
        num_cores=NC, num_subcores=NS)
    sc_params = pltpu.CompilerParams(needs_layout_passes=False)

    gather = pl.kernel(
        _sc_gather_body,
        out_type=(jax.ShapeDtypeStruct((ES, D), _f32),
                  jax.ShapeDtypeStruct((8, ES), _f32)),
        mesh=mesh,
        scratch_types=[
            pltpu.VMEM((2, CHUNK), _i32),
            pltpu.VMEM((2, CHUNK), _i32),
            pltpu.VMEM((2, CHUNK, D), _f32),
            pltpu.VMEM((2, 8, CHUNK), _f32),
            pltpu.VMEM((CD * N,), _f32),
            pltpu.SemaphoreType.DMA,
            pltpu.SemaphoreType.DMA,
            pltpu.SemaphoreType.DMA,
            pltpu.SemaphoreType.DMA,
            pltpu.SemaphoreType.DMA,
        ],
        compiler_params=sc_params,
    )

    eblk = 3200
    edge_mlp = pl.pallas_call(
        _edge_body,
        grid=(ES // eblk,),
        in_specs=[
            pl.BlockSpec((eblk, D), lambda i: (i, 0)),
            pl.BlockSpec((8, eblk), lambda i: (0, i)),
            pl.BlockSpec((1, D), lambda i: (0, 0)),
            pl.BlockSpec((1, D), lambda i: (0, 0)),
            pl.BlockSpec((D, D), lambda i: (0, 0)),
            pl.BlockSpec((1, D), lambda i: (0, 0)),
            pl.BlockSpec((D, D), lambda i: (0, 0)),
            pl.BlockSpec((1, D), lambda i: (0, 0)),
            pl.BlockSpec((1, D), lambda i: (0, 0)),
        ],
        out_specs=(pl.BlockSpec((eblk, D), lambda i: (i, 0)),
                   pl.BlockSpec((8, eblk), lambda i: (0, i))),
        out_shape=(jax.ShapeDtypeStruct((ES, D), _f32),
                   jax.ShapeDtypeStruct((8, ES), _f32)),
    )

    scatter = pl.kernel(
        _sc_scatter_body,
        out_type=(jax.ShapeDtypeStruct((NC, NP, D), _f32),
                  jax.ShapeDtypeStruct((NC, 4 * NP), _f32)),
        mesh=mesh,
        scratch_types=[
            pltpu.VMEM((2, CHUNK), _i32),
            pltpu.VMEM((2, 4, CHUNK), _i32),
            pltpu.VMEM((2, CHUNK, D), _f32),
            pltpu.VMEM((2, 8, CHUNK), _f32),
            pltpu.VMEM((BR, D), _f32),
            pltpu.VMEM((TPS,), _f32),
            pltpu.VMEM_SHARED((N, D), _f32),
            pltpu.VMEM_SHARED((4 * NP,), _f32),
            pltpu.SemaphoreType.DMA,
            pltpu.SemaphoreType.DMA,
            pltpu.SemaphoreType.DMA,
            pltpu.SemaphoreType.DMA,
        ],
        compiler_params=sc_params,
    )

    z128 = jnp.zeros((BR, D), _f32)
    z1 = jnp.zeros((TPS,), _f32)
    accs = []
    txs = []
    for sl in range(NSL):
        i0s = lax.slice_in_dim(ei0, sl * ES, (sl + 1) * ES)
        i1s = lax.slice_in_dim(ei1, sl * ES, (sl + 1) * ES)
        g0, cdt = gather(t0, t1, i0s, i1s, cp1)
        ef, tt = edge_mlp(g0, cdt, We1[2 * D:2 * D + 1], be1[None], We2,
                          be2[None], Wc1, bc1[None], Wc2.T)
        accd, t_out = scatter(ef, tt, i0s, z128, z1)
        accs.append(accd)
        txs.append(jnp.reshape(t_out, (8, NP)))

    nblk = 1280
    node_mlp = pl.pallas_call(
        _node_body,
        grid=(NP // nblk,),
        in_specs=[
            pl.BlockSpec((nblk, D), lambda i: (i, 0)),
            pl.BlockSpec((8, nblk), lambda i: (0, i)),
            pl.BlockSpec((NC, nblk, D), lambda i: (0, i, 0)),
            pl.BlockSpec((NC, nblk, D), lambda i: (0, i, 0)),
            pl.BlockSpec((8, nblk), lambda i: (0, i)),
            pl.BlockSpec((8, nblk), lambda i: (0, i)),
            pl.BlockSpec((D, D), lambda i: (0, 0)),
            pl.BlockSpec((D, D), lambda i: (0, 0)),
            pl.BlockSpec((1, D), lambda i: (0, 0)),
            pl.BlockSpec((D, D), lambda i: (0, 0)),
            pl.BlockSpec((1, D), lambda i: (0, 0)),
        ],
        out_specs=(pl.BlockSpec((nblk, D), lambda i: (i, 0)),
                   pl.BlockSpec((8, nblk), lambda i: (0, i))),
        out_shape=(jax.ShapeDtypeStruct((NP, D), _f32),
                   jax.ShapeDtypeStruct((8, NP), _f32)),
    )
    nout, coutT = node_mlp(nfp, cpT, accs[0], accs[1], txs[0], txs[1],
                           Wn1[:D], Wn1[D:], bn1[None], Wn2, bn2[None])
    return nout[:N], coutT[:CD, :N].T
